# baseline (speedup 1.0000x reference)
import functools

import jax
import jax.numpy as jnp
from jax import lax
from jax.experimental import pallas as pl
from jax.experimental.pallas import tpu as pltpu
from jax.experimental.pallas import tpu_sc as plsc

NUM_WEIGHTS = 512
PADDING_IDX = 1
LANES = 16
NUM_WORKERS = 32


def _build(T, D, V):
    R = V // NUM_WORKERS
    mesh = plsc.VectorSubcoreMesh(core_axis_name="c", subcore_axis_name="s")

    @functools.partial(
        pl.kernel,
        mesh=mesh,
        compiler_params=pltpu.CompilerParams(needs_layout_passes=False),
        out_type=jax.ShapeDtypeStruct((V, D), jnp.float32),
        scratch_types=[
            pltpu.VMEM((T,), jnp.int32),
            pltpu.VMEM((D,), jnp.float32),
            pltpu.VMEM((R, D), jnp.float32),
            pltpu.VMEM((LANES,), jnp.float32),
        ],
    )
    def k(grad_hbm, idx_hbm, out_hbm, idx_v, row_v, acc_v, cnt_v):
        wid = lax.axis_index("s") * 2 + lax.axis_index("c")
        base = wid * R

        pltpu.sync_copy(idx_hbm.at[0], idx_v)

        for i in range(R):
            def zero_body(j, _, i=i):
                acc_v[i, pl.ds(j * LANES, LANES)] = jnp.zeros(
                    (LANES,), jnp.float32)
                return 0
            lax.fori_loop(0, D // LANES, zero_body, 0, unroll=8)

        row_ids = base + lax.broadcasted_iota(jnp.int32, (LANES,), 0)
        cnt_v[...] = jnp.zeros((LANES,), jnp.float32)

        def grp_body(g, _):
            rvec = idx_v[pl.ds(g * LANES, LANES)]
            owned_vec = ((rvec >= base) & (rvec < base + R)
                         & (rvec != PADDING_IDX))

            any_owned = lax.sort(jnp.where(owned_vec, 1, 0))[LANES - 1] == 1

            @pl.when(any_owned)
            def _(g=g, rvec=rvec):
                cnt16 = cnt_v[...]
                for lane in range(LANES):
                    r = rvec[lane]
                    t = g * LANES + lane
                    cnt16 = cnt16 + jnp.where(row_ids == r, 1.0, 0.0)
                    matched = ((r >= base) & (r < base + R)
                               & (r != PADDING_IDX))

                    @pl.when(matched)
                    def _(r=r, t=t):
                        pltpu.sync_copy(grad_hbm.at[0, t], row_v)
                        loc = r - base
                        for j in range(D // LANES):
                            sl = pl.ds(j * LANES, LANES)
                            acc_v[loc, sl] = acc_v[loc, sl] + row_v[sl]
                cnt_v[...] = cnt16
            return 0
        lax.fori_loop(0, T // LANES, grp_body, 0)
        cnt16 = cnt_v[...]

        inv16 = 1.0 / jnp.maximum(cnt16, 1.0)
        for i in range(R):
            @pl.when(cnt16[i] > 1.0)
            def _(i=i):
                iv = inv16[i]

                def sc_body(j, _, i=i, iv=iv):
                    sl = pl.ds(j * LANES, LANES)
                    acc_v[i, sl] = acc_v[i, sl] * iv
                    return 0
                lax.fori_loop(0, D // LANES, sc_body, 0, unroll=6)

        pltpu.sync_copy(acc_v, out_hbm.at[pl.ds(base, R)])

    return k


def kernel(grad_output, index):
    T = index.shape[0] * index.shape[1]
    D = grad_output.shape[-1]
    idx = index.astype(jnp.int32)
    return _build(T, D, NUM_WEIGHTS)(grad_output, idx)


# speedup vs baseline: 1.0753x; 1.0753x over previous
import functools

import jax
import jax.numpy as jnp
from jax import lax
from jax.experimental import pallas as pl
from jax.experimental.pallas import tpu as pltpu
from jax.experimental.pallas import tpu_sc as plsc

NUM_WEIGHTS = 512
PADDING_IDX = 1
LANES = 16
NUM_WORKERS = 32


def _build(T, D, V):
    R = V // NUM_WORKERS
    mesh = plsc.VectorSubcoreMesh(core_axis_name="c", subcore_axis_name="s")

    @functools.partial(
        pl.kernel,
        mesh=mesh,
        compiler_params=pltpu.CompilerParams(needs_layout_passes=False),
        out_type=jax.ShapeDtypeStruct((V, D), jnp.float32),
        scratch_types=[
            pltpu.VMEM((T,), jnp.int32),
            pltpu.VMEM((T, D), jnp.float32),
            pltpu.VMEM((R, D), jnp.float32),
            pltpu.VMEM((LANES,), jnp.float32),
            pltpu.SemaphoreType.DMA,
        ],
    )
    def k(grad_hbm, idx_hbm, out_hbm, idx_v, slots_v, acc_v, cnt_v, sem):
        wid = lax.axis_index("s") * 2 + lax.axis_index("c")
        base = wid * R

        pltpu.sync_copy(idx_hbm.at[0], idx_v)

        for i in range(R):
            def zero_body(j, _, i=i):
                acc_v[i, pl.ds(j * LANES, LANES)] = jnp.zeros(
                    (LANES,), jnp.float32)
                return 0
            lax.fori_loop(0, D // LANES, zero_body, 0, unroll=8)

        row_ids = base + lax.broadcasted_iota(jnp.int32, (LANES,), 0)
        cnt_v[...] = jnp.zeros((LANES,), jnp.float32)

        def any16(mask):
            return lax.sort(jnp.where(mask, 1, 0))[LANES - 1] == 1

        def fire_body(g, _):
            rvec = idx_v[pl.ds(g * LANES, LANES)]
            owned_vec = ((rvec >= base) & (rvec < base + R)
                         & (rvec != PADDING_IDX))

            @pl.when(any16(owned_vec))
            def _(g=g, rvec=rvec):
                cnt16 = cnt_v[...]
                for lane in range(LANES):
                    r = rvec[lane]
                    t = g * LANES + lane
                    cnt16 = cnt16 + jnp.where(row_ids == r, 1.0, 0.0)
                    matched = ((r >= base) & (r < base + R)
                               & (r != PADDING_IDX))

                    @pl.when(matched)
                    def _(r=r, t=t):
                        pltpu.async_copy(grad_hbm.at[0, t], slots_v.at[t],
                                         sem)
                cnt_v[...] = cnt16
            return 0
        lax.fori_loop(0, T // LANES, fire_body, 0)

        def drain_body(g, _):
            rvec = idx_v[pl.ds(g * LANES, LANES)]
            owned_vec = ((rvec >= base) & (rvec < base + R)
                         & (rvec != PADDING_IDX))

            @pl.when(any16(owned_vec))
            def _(g=g, rvec=rvec):
                for lane in range(LANES):
                    r = rvec[lane]
                    t = g * LANES + lane
                    matched = ((r >= base) & (r < base + R)
                               & (r != PADDING_IDX))

                    @pl.when(matched)
                    def _(r=r, t=t):
                        pltpu.make_async_copy(grad_hbm.at[0, t],
                                              slots_v.at[t], sem).wait()
                        loc = r - base
                        for j in range(D // LANES):
                            sl = pl.ds(j * LANES, LANES)
                            acc_v[loc, sl] = (acc_v[loc, sl]
                                              + slots_v[t, sl])
            return 0
        lax.fori_loop(0, T // LANES, drain_body, 0)
        cnt16 = cnt_v[...]

        inv16 = 1.0 / jnp.maximum(cnt16, 1.0)
        for i in range(R):
            @pl.when(cnt16[i] > 1.0)
            def _(i=i):
                iv = inv16[i]

                def sc_body(j, _, i=i, iv=iv):
                    sl = pl.ds(j * LANES, LANES)
                    acc_v[i, sl] = acc_v[i, sl] * iv
                    return 0
                lax.fori_loop(0, D // LANES, sc_body, 0, unroll=6)

        pltpu.sync_copy(acc_v, out_hbm.at[pl.ds(base, R)])

    return k


def kernel(grad_output, index):
    T = index.shape[0] * index.shape[1]
    D = grad_output.shape[-1]
    idx = index.astype(jnp.int32)
    return _build(T, D, NUM_WEIGHTS)(grad_output, idx)


# speedup vs baseline: 1.1931x; 1.1095x over previous
import functools

import jax
import jax.numpy as jnp
from jax import lax
from jax.experimental import pallas as pl
from jax.experimental.pallas import tpu as pltpu
from jax.experimental.pallas import tpu_sc as plsc

NUM_WEIGHTS = 512
PADDING_IDX = 1
LANES = 16
NUM_WORKERS = 32


def _build(T, D, V):
    R = V // NUM_WORKERS
    mesh = plsc.VectorSubcoreMesh(core_axis_name="c", subcore_axis_name="s")

    @functools.partial(
        pl.kernel,
        mesh=mesh,
        compiler_params=pltpu.CompilerParams(needs_layout_passes=False),
        out_type=jax.ShapeDtypeStruct((V, D), jnp.float32),
        scratch_types=[
            pltpu.VMEM((T,), jnp.int32),
            pltpu.VMEM((T, D), jnp.float32),
            pltpu.VMEM((R, D), jnp.float32),
            pltpu.VMEM((LANES,), jnp.float32),
            pltpu.SemaphoreType.DMA,
        ],
    )
    def k(grad_hbm, idx_hbm, out_hbm, idx_v, slots_v, acc_v, cnt_v, sem):
        wid = lax.axis_index("s") * 2 + lax.axis_index("c")
        base = wid * R

        pltpu.sync_copy(idx_hbm.at[0], idx_v)

        for i in range(R):
            def zero_body(j, _, i=i):
                acc_v[i, pl.ds(j * LANES, LANES)] = jnp.zeros(
                    (LANES,), jnp.float32)
                return 0
            lax.fori_loop(0, D // LANES, zero_body, 0, unroll=8)

        row_ids = base + lax.broadcasted_iota(jnp.int32, (LANES,), 0)
        cnt_v[...] = jnp.zeros((LANES,), jnp.float32)

        def any16(mask):
            return plsc.all_reduce_population_count(mask)[0] > 0

        def fire_body(g, _):
            rvec = idx_v[pl.ds(g * LANES, LANES)]
            owned_vec = ((rvec >= base) & (rvec < base + R)
                         & (rvec != PADDING_IDX))

            @pl.when(any16(owned_vec))
            def _(g=g, rvec=rvec, owned_vec=owned_vec):
                ivec = jnp.where(owned_vec, 1, 0)
                cnt16 = cnt_v[...]
                for lane in range(LANES):
                    r = rvec[lane]
                    t = g * LANES + lane
                    cnt16 = cnt16 + jnp.where(row_ids == r, 1.0, 0.0)

                    @pl.when(ivec[lane] == 1)
                    def _(r=r, t=t):
                        pltpu.async_copy(grad_hbm.at[0, t], slots_v.at[t],
                                         sem)
                cnt_v[...] = cnt16
            return 0
        lax.fori_loop(0, T // LANES, fire_body, 0)

        def drain_body(g, _):
            rvec = idx_v[pl.ds(g * LANES, LANES)]
            owned_vec = ((rvec >= base) & (rvec < base + R)
                         & (rvec != PADDING_IDX))

            @pl.when(any16(owned_vec))
            def _(g=g, rvec=rvec, owned_vec=owned_vec):
                ivec = jnp.where(owned_vec, 1, 0)
                for lane in range(LANES):
                    r = rvec[lane]
                    t = g * LANES + lane

                    @pl.when(ivec[lane] == 1)
                    def _(r=r, t=t):
                        pltpu.make_async_copy(grad_hbm.at[0, t],
                                              slots_v.at[t], sem).wait()
                        loc = r - base
                        for j in range(D // LANES):
                            sl = pl.ds(j * LANES, LANES)
                            acc_v[loc, sl] = (acc_v[loc, sl]
                                              + slots_v[t, sl])
            return 0
        lax.fori_loop(0, T // LANES, drain_body, 0)
        cnt16 = cnt_v[...]

        inv16 = 1.0 / jnp.maximum(cnt16, 1.0)
        for i in range(R):
            @pl.when(cnt16[i] > 1.0)
            def _(i=i):
                iv = inv16[i]

                def sc_body(j, _, i=i, iv=iv):
                    sl = pl.ds(j * LANES, LANES)
                    acc_v[i, sl] = acc_v[i, sl] * iv
                    return 0
                lax.fori_loop(0, D // LANES, sc_body, 0, unroll=6)

        pltpu.sync_copy(acc_v, out_hbm.at[pl.ds(base, R)])

    return k


def kernel(grad_output, index):
    T = index.shape[0] * index.shape[1]
    D = grad_output.shape[-1]
    idx = index.astype(jnp.int32)
    return _build(T, D, NUM_WEIGHTS)(grad_output, idx)


# speedup vs baseline: 1.1973x; 1.0036x over previous
import functools

import jax
import jax.numpy as jnp
from jax import lax
from jax.experimental import pallas as pl
from jax.experimental.pallas import tpu as pltpu
from jax.experimental.pallas import tpu_sc as plsc

NUM_WEIGHTS = 512
PADDING_IDX = 1
LANES = 16
NUM_WORKERS = 32


def _build(T, D, V):
    R = V // NUM_WORKERS
    mesh = plsc.VectorSubcoreMesh(core_axis_name="c", subcore_axis_name="s")

    @functools.partial(
        pl.kernel,
        mesh=mesh,
        compiler_params=pltpu.CompilerParams(needs_layout_passes=False),
        out_type=jax.ShapeDtypeStruct((V, D), jnp.float32),
        scratch_types=[
            pltpu.VMEM((T,), jnp.int32),
            pltpu.VMEM((T, D), jnp.float32),
            pltpu.VMEM((R, D), jnp.float32),
            pltpu.VMEM((LANES,), jnp.float32),
            pltpu.SemaphoreType.DMA,
        ],
    )
    def k(grad_hbm, idx_hbm, out_hbm, idx_v, slots_v, acc_v, cnt_v, sem):
        wid = lax.axis_index("s") * 2 + lax.axis_index("c")
        base = wid * R

        pltpu.sync_copy(idx_hbm.at[0], idx_v)

        for i in range(R):
            def zero_body(j, _, i=i):
                acc_v[i, pl.ds(j * LANES, LANES)] = jnp.zeros(
                    (LANES,), jnp.float32)
                return 0
            lax.fori_loop(0, D // LANES, zero_body, 0, unroll=8)

        row_ids = base + lax.broadcasted_iota(jnp.int32, (LANES,), 0)
        cnt_v[...] = jnp.zeros((LANES,), jnp.float32)

        def any16(mask):
            return plsc.all_reduce_population_count(mask)[0] > 0

        def fire_body(g, _):
            rvec = idx_v[pl.ds(g * LANES, LANES)]
            owned_vec = ((rvec >= base) & (rvec < base + R)
                         & (rvec != PADDING_IDX))

            @pl.when(any16(owned_vec))
            def _(g=g, rvec=rvec, owned_vec=owned_vec):
                ivec = jnp.where(owned_vec, 1, 0)
                cnt16 = cnt_v[...]
                for lane in range(LANES):
                    r = rvec[lane]
                    t = g * LANES + lane
                    cnt16 = cnt16 + jnp.where(row_ids == r, 1.0, 0.0)

                    @pl.when(ivec[lane] == 1)
                    def _(r=r, t=t):
                        pltpu.async_copy(grad_hbm.at[0, t], slots_v.at[t],
                                         sem)
                cnt_v[...] = cnt16
            return 0
        lax.fori_loop(0, T // LANES, fire_body, 0)

        def drain_body(g, _):
            rvec = idx_v[pl.ds(g * LANES, LANES)]
            owned_vec = ((rvec >= base) & (rvec < base + R)
                         & (rvec != PADDING_IDX))

            @pl.when(any16(owned_vec))
            def _(g=g, rvec=rvec, owned_vec=owned_vec):
                ivec = jnp.where(owned_vec, 1, 0)
                for lane in range(LANES):
                    r = rvec[lane]
                    t = g * LANES + lane

                    @pl.when(ivec[lane] == 1)
                    def _(r=r, t=t):
                        pltpu.make_async_copy(grad_hbm.at[0, t],
                                              slots_v.at[t], sem).wait()
                        loc = r - base

                        def acc_body(j, _, loc=loc, t=t):
                            sl = pl.ds(j * LANES, LANES)
                            acc_v[loc, sl] = (acc_v[loc, sl]
                                              + slots_v[t, sl])
                            return 0
                        lax.fori_loop(0, D // LANES, acc_body, 0, unroll=4)
            return 0
        lax.fori_loop(0, T // LANES, drain_body, 0)
        cnt16 = cnt_v[...]

        inv16 = 1.0 / jnp.maximum(cnt16, 1.0)
        for i in range(R):
            @pl.when(cnt16[i] > 1.0)
            def _(i=i):
                iv = inv16[i]

                def sc_body(j, _, i=i, iv=iv):
                    sl = pl.ds(j * LANES, LANES)
                    acc_v[i, sl] = acc_v[i, sl] * iv
                    return 0
                lax.fori_loop(0, D // LANES, sc_body, 0, unroll=3)

        pltpu.sync_copy(acc_v, out_hbm.at[pl.ds(base, R)])

    return k


def kernel(grad_output, index):
    T = index.shape[0] * index.shape[1]
    D = grad_output.shape[-1]
    idx = index.astype(jnp.int32)
    return _build(T, D, NUM_WEIGHTS)(grad_output, idx)
